# Initial kernel scaffold; baseline (speedup 1.0000x reference)
import jax
import jax.numpy as jnp
from jax.experimental import pallas as pl


def kernel(tensor, values, indices):
    raise NotImplementedError("write your pallas kernel here")


# speedup vs baseline: 4.0333x; 4.0333x over previous
import functools

import jax
import jax.numpy as jnp
from jax import lax
from jax.experimental import pallas as pl
from jax.experimental.pallas import tpu as pltpu
from jax.experimental.pallas import tpu_sc as plsc

NUMEL = 4096 * 4096
NC = 2
NS = 16
NW = NC * NS
C = 32768
NCH = NUMEL // C
CPT = NCH // NW
B = 1024
L = 16


def _body(flat_hbm, idx_hbm, val_hbm, st_hbm, en_hbm, out_hbm,
          st_v, en_v, chunk_v, idx_v, val_v):
  cid = lax.axis_index("c")
  sid = lax.axis_index("s")
  wid = sid * NC + cid
  cbase = wid * CPT

  pltpu.sync_copy(st_hbm.at[pl.ds(wid * CPT, L)], st_v.at[pl.ds(0, L)])
  pltpu.sync_copy(en_hbm.at[pl.ds(wid * CPT, L)], en_v.at[pl.ds(0, L)])

  def chunk_body(c, _):
    start = st_v[pl.ds(c, L)][0]
    end = en_v[pl.ds(c, L)][0]
    gb = pl.multiple_of((cbase + c) * C, C)

    pltpu.sync_copy(flat_hbm.at[pl.ds(gb, C)], chunk_v)

    s8 = pl.multiple_of(start & -8, 8)
    nb = (end - s8 + (B - 1)) // B

    def blk(b, __):
      off = pl.multiple_of(s8 + b * B, 8)
      pltpu.sync_copy(idx_hbm.at[pl.ds(off, B)], idx_v)
      pltpu.sync_copy(val_hbm.at[pl.ds(off, B)], val_v)
      for j in range(B // L):
        iv = idx_v[pl.ds(j * L, L)]
        vv = val_v[pl.ds(j * L, L)]
        loc = iv - gb
        inb = (loc >= 0) & (loc < C)
        lc = jnp.minimum(jnp.maximum(loc, 0), C - 1)
        vz = jnp.where(inb, vv, 0.0)
        plsc.addupdate_scatter(chunk_v, [lc], vz)
      return 0

    lax.fori_loop(0, nb, blk, 0)
    pltpu.sync_copy(chunk_v, out_hbm.at[pl.ds(gb, C)])
    return 0

  lax.fori_loop(0, CPT, chunk_body, 0)


_sc_call = functools.partial(
    pl.kernel,
    out_type=jax.ShapeDtypeStruct((NUMEL,), jnp.float32),
    mesh=plsc.VectorSubcoreMesh(
        core_axis_name="c", subcore_axis_name="s",
        num_cores=NC, num_subcores=NS),
    compiler_params=pltpu.CompilerParams(needs_layout_passes=False),
    scratch_types=[
        pltpu.VMEM((2 * L,), jnp.int32),
        pltpu.VMEM((2 * L,), jnp.int32),
        pltpu.VMEM((C,), jnp.float32),
        pltpu.VMEM((B,), jnp.int32),
        pltpu.VMEM((B,), jnp.float32),
    ],
)(_body)


def kernel(tensor, values, indices):
  idx32 = indices.astype(jnp.int32)
  flat = tensor.reshape(-1)
  bounds = jnp.arange(0, NUMEL + 1, C, dtype=jnp.int32)
  pos = jnp.searchsorted(idx32, bounds, side="left").astype(jnp.int32)
  starts = pos[:-1]
  ends = pos[1:]
  idx_p = jnp.concatenate([idx32, jnp.full((B,), NUMEL, jnp.int32)])
  val_p = jnp.concatenate(
      [values.astype(jnp.float32), jnp.zeros((B,), jnp.float32)])
  out = _sc_call(flat, idx_p, val_p, starts, ends)
  return out.reshape(tensor.shape)


# speedup vs baseline: 4.7163x; 1.1693x over previous
import functools

import jax
import jax.numpy as jnp
from jax import lax
from jax.experimental import pallas as pl
from jax.experimental.pallas import tpu as pltpu
from jax.experimental.pallas import tpu_sc as plsc

NUMEL = 4096 * 4096
NC = 2
NS = 16
NW = NC * NS
C = 32768
NCH = NUMEL // C
CPT = NCH // NW
B = 1024
L = 16


def _scatter_block(chunk_ref, idx_ref, val_ref, gb):
  for j in range(B // L):
    iv = idx_ref[pl.ds(j * L, L)]
    vv = val_ref[pl.ds(j * L, L)]
    loc = iv - gb
    inb = (loc >= 0) & (loc < C)
    lc = jnp.minimum(jnp.maximum(loc, 0), C - 1)
    vz = jnp.where(inb, vv, 0.0)
    plsc.addupdate_scatter(chunk_ref, [lc], vz)


def _body(flat_hbm, idx_hbm, val_hbm, st_hbm, en_hbm, out_hbm,
          st_v, en_v, cv0, cv1, ix0, ix1, vl0, vl1,
          isem0, isem1, osem0, osem1, xsem0, xsem1, wsem0, wsem1):
  cid = lax.axis_index("c")
  sid = lax.axis_index("s")
  wid = sid * NC + cid
  cbase = wid * CPT

  bufs = (cv0, cv1)
  ixs = (ix0, ix1)
  vls = (vl0, vl1)
  isems = (isem0, isem1)
  osems = (osem0, osem1)
  xsems = (xsem0, xsem1)
  wsems = (wsem0, wsem1)

  pltpu.sync_copy(st_hbm.at[pl.ds(wid * CPT, L)], st_v.at[pl.ds(0, L)])
  pltpu.sync_copy(en_hbm.at[pl.ds(wid * CPT, L)], en_v.at[pl.ds(0, L)])

  def s8_of(c):
    return pl.multiple_of(st_v[pl.ds(c, L)][0] & -8, 8)

  def gb_of(c):
    return pl.multiple_of((cbase + c) * C, C)

  def start_in(c, p):
    pltpu.async_copy(flat_hbm.at[pl.ds(gb_of(c), C)], bufs[p], isems[p])
    s8 = s8_of(c)
    pltpu.async_copy(idx_hbm.at[pl.ds(s8, B)], ixs[p], xsems[p])
    pltpu.async_copy(val_hbm.at[pl.ds(s8, B)], vls[p], wsems[p])

  def wait_in(p):
    pltpu.make_async_copy(flat_hbm.at[pl.ds(0, C)], bufs[p], isems[p]).wait()
    pltpu.make_async_copy(idx_hbm.at[pl.ds(0, B)], ixs[p], xsems[p]).wait()
    pltpu.make_async_copy(val_hbm.at[pl.ds(0, B)], vls[p], wsems[p]).wait()

  def wait_out(p):
    pltpu.make_async_copy(bufs[p], out_hbm.at[pl.ds(0, C)], osems[p]).wait()

  start_in(0, 0)

  def pair_body(g, _):
    for p in (0, 1):
      c = g * 2 + p
      q = 1 - p
      @pl.when(c >= 1)
      def _():
        wait_out(q)

      @pl.when(c + 1 < CPT)
      def _():
        start_in(c + 1, q)

      wait_in(p)

      gb = gb_of(c)
      s8 = s8_of(c)
      end = en_v[pl.ds(c, L)][0]
      nb = (end - s8 + (B - 1)) // B

      @pl.when(nb >= 1)
      def _():
        _scatter_block(bufs[p], ixs[p], vls[p], gb)

      def blk(b, __):
        off = pl.multiple_of(s8 + b * B, 8)
        pltpu.sync_copy(idx_hbm.at[pl.ds(off, B)], ixs[p])
        pltpu.sync_copy(val_hbm.at[pl.ds(off, B)], vls[p])
        _scatter_block(bufs[p], ixs[p], vls[p], gb)
        return 0

      lax.fori_loop(1, nb, blk, 0)
      pltpu.async_copy(bufs[p], out_hbm.at[pl.ds(gb, C)], osems[p])
    return 0

  lax.fori_loop(0, CPT // 2, pair_body, 0)
  wait_out(1)


_sc_call = functools.partial(
    pl.kernel,
    out_type=jax.ShapeDtypeStruct((NUMEL,), jnp.float32),
    mesh=plsc.VectorSubcoreMesh(
        core_axis_name="c", subcore_axis_name="s",
        num_cores=NC, num_subcores=NS),
    compiler_params=pltpu.CompilerParams(needs_layout_passes=False),
    scratch_types=[
        pltpu.VMEM((2 * L,), jnp.int32),
        pltpu.VMEM((2 * L,), jnp.int32),
        pltpu.VMEM((C,), jnp.float32),
        pltpu.VMEM((C,), jnp.float32),
        pltpu.VMEM((B,), jnp.int32),
        pltpu.VMEM((B,), jnp.int32),
        pltpu.VMEM((B,), jnp.float32),
        pltpu.VMEM((B,), jnp.float32),
        pltpu.SemaphoreType.DMA,
        pltpu.SemaphoreType.DMA,
        pltpu.SemaphoreType.DMA,
        pltpu.SemaphoreType.DMA,
        pltpu.SemaphoreType.DMA,
        pltpu.SemaphoreType.DMA,
        pltpu.SemaphoreType.DMA,
        pltpu.SemaphoreType.DMA,
    ],
)(_body)


def kernel(tensor, values, indices):
  idx32 = indices.astype(jnp.int32)
  flat = tensor.reshape(-1)
  bounds = jnp.arange(0, NUMEL + 1, C, dtype=jnp.int32)
  pos = jnp.searchsorted(idx32, bounds, side="left").astype(jnp.int32)
  starts = pos[:-1]
  ends = pos[1:]
  idx_p = jnp.concatenate([idx32, jnp.full((B,), NUMEL, jnp.int32)])
  val_p = jnp.concatenate(
      [values.astype(jnp.float32), jnp.zeros((B,), jnp.float32)])
  out = _sc_call(flat, idx_p, val_p, starts, ends)
  return out.reshape(tensor.shape)
